# baseline (speedup 1.0000x reference)
import functools

import jax
import jax.numpy as jnp
from jax import lax
from jax.experimental import pallas as pl
from jax.experimental.pallas import tpu as pltpu
from jax.experimental.pallas import tpu_sc as plsc

NC = 2
NS = 16
LANES = 16
BSZ = 4
NSTEP = 2
BPC = BSZ // NC

CHUNK = 4000


@functools.partial(jax.jit, static_argnums=(6, 7, 8))
def _fused_call(e_flat, sub, obj, p_flat, w_flat, f_flat,
                nent_pad, ept, e_pad):
    nchunk = ept // CHUNK
    rpt = nent_pad // NS

    mesh = plsc.VectorSubcoreMesh(
        core_axis_name="c", subcore_axis_name="s",
        num_cores=NC, num_subcores=NS)

    @functools.partial(
        pl.kernel,
        out_type=jax.ShapeDtypeStruct((BSZ * nent_pad,), jnp.float32),
        mesh=mesh,
        scratch_types=(
            [pltpu.VMEM_SHARED((nent_pad,), jnp.float32)
             for _ in range(2 * BPC)]
            + [pltpu.VMEM((CHUNK,), jnp.int32) for _ in range(4)]
            + [pltpu.VMEM((CHUNK,), jnp.float32) for _ in range(4 * BPC)]
            + [pltpu.VMEM((rpt,), jnp.float32) for _ in range(3 + BPC)]
            + [pltpu.VMEM((NSTEP * BSZ * LANES,), jnp.float32)]
            + [pltpu.SemaphoreType.DMA for _ in range(5)]
        ),
    )
    def fused(e_hbm, sub_hbm, obj_hbm, p_hbm, w_hbm, f_hbm, out_hbm,
              ta, tb, aa, ab,
              sub0, sub1, obj0, obj1,
              pc00, pc01, pc10, pc11,
              g00, g01, g10, g11,
              stage, zbuf, fbuf, e1w0, e1w1, wbuf,
              sin0, sin1, sg, ss0, ss1):
        tabs = (ta, tb)
        accs = (aa, ab)
        subc = (sub0, sub1)
        objc = (obj0, obj1)
        pcs = ((pc00, pc01), (pc10, pc11))
        gs = ((g00, g01), (g10, g11))
        e1w = (e1w0, e1w1)
        sem_in = (sin0, sin1)
        sem_s = (ss0, ss1)
        c = lax.axis_index("c")
        s = lax.axis_index("s")
        ent0 = s * rpt

        def issue_inputs(t, j, slot):
            base = s * ept + j * CHUNK
            pltpu.async_copy(sub_hbm.at[pl.ds(base, CHUNK)], subc[slot],
                             sem_in[slot])
            pltpu.async_copy(obj_hbm.at[pl.ds(base, CHUNK)], objc[slot],
                             sem_in[slot])
            for b in range(BPC):
                boff = ((t * NC + c) * BPC + b) * e_pad
                pltpu.async_copy(p_hbm.at[pl.ds(boff + base, CHUNK)],
                                 pcs[slot][b], sem_in[slot])

        def wait_inputs(slot):
            pltpu.make_async_copy(sub_hbm.at[pl.ds(0, CHUNK)], subc[slot],
                                  sem_in[slot]).wait()
            pltpu.make_async_copy(obj_hbm.at[pl.ds(0, CHUNK)], objc[slot],
                                  sem_in[slot]).wait()
            for b in range(BPC):
                pltpu.make_async_copy(p_hbm.at[pl.ds(0, CHUNK)],
                                      pcs[slot][b], sem_in[slot]).wait()

        def wait_scatters(slot):
            for b in range(BPC):
                pltpu.make_async_copy(gs[slot][b], accs[b].at[objc[slot]],
                                      sem_s[slot]).wait()

        def zfill(v, carry):
            zbuf[pl.ds(v * LANES, LANES)] = jnp.zeros((LANES,), jnp.float32)
            return carry
        lax.fori_loop(0, rpt // LANES, zfill, 0)
        pltpu.sync_copy(w_hbm, wbuf)
        for b in range(BPC):
            gb = c * BPC + b
            pltpu.sync_copy(e_hbm.at[pl.ds(gb * nent_pad + ent0, rpt)],
                            stage)
            pltpu.sync_copy(stage, tabs[b].at[pl.ds(ent0, rpt)])
            pltpu.sync_copy(zbuf, accs[b].at[pl.ds(ent0, rpt)])
        plsc.subcore_barrier()

        for t in range(NSTEP):
            issue_inputs(t, 0, 0)

            def chunk_body(i, carry):
                slot = lax.rem(i, 2)

                def slot_body(sl, ot):
                    @pl.when(i > 0)
                    def _():
                        wait_scatters(ot)
                    nxt = jnp.minimum(i + 1, nchunk - 1)
                    issue_inputs(t, nxt, ot)
                    wait_inputs(sl)
                    for b in range(BPC):
                        pltpu.async_copy(tabs[b].at[subc[sl]], gs[sl][b],
                                         sg)
                    for b in range(BPC):
                        pltpu.make_async_copy(tabs[b].at[subc[sl]],
                                              gs[sl][b], sg).wait()
                    def mul_body(v, carry2):
                        vsl = pl.ds(v * LANES, LANES)
                        for b in range(BPC):
                            gs[sl][b][vsl] = gs[sl][b][vsl] * pcs[sl][b][vsl]
                        return carry2
                    lax.fori_loop(0, CHUNK // LANES, mul_body, 0)
                    for b in range(BPC):
                        pltpu.async_copy(gs[sl][b], accs[b].at[objc[sl]],
                                         sem_s[sl], add=True)

                @pl.when(slot == 0)
                def _():
                    slot_body(0, 1)

                @pl.when(slot == 1)
                def _():
                    slot_body(1, 0)
                return carry
            lax.fori_loop(0, nchunk, chunk_body, 0)

            wait_scatters((nchunk - 1) % 2)
            wait_inputs(nchunk % 2)
            plsc.subcore_barrier()

            for b in range(BPC):
                pltpu.sync_copy(accs[b].at[pl.ds(ent0, rpt)], stage)
                if t == 0:
                    w0 = wbuf[pl.ds((0 * BSZ + c * BPC + b) * LANES, LANES)]

                    def norm0(v, carry, _b=b, _w0=w0):
                        vsl = pl.ds(v * LANES, LANES)
                        x = stage[vsl]
                        x = x / jnp.maximum(x, 1.0)
                        stage[vsl] = x
                        e1w[_b][vsl] = x * _w0
                        return carry
                    lax.fori_loop(0, rpt // LANES, norm0, 0)
                    pltpu.sync_copy(stage, tabs[b].at[pl.ds(ent0, rpt)])
                    pltpu.sync_copy(zbuf, accs[b].at[pl.ds(ent0, rpt)])
                else:
                    gb = c * BPC + b
                    pltpu.sync_copy(f_hbm.at[pl.ds(gb * nent_pad + ent0,
                                                   rpt)], fbuf)
                    w1 = wbuf[pl.ds((1 * BSZ + gb) * LANES, LANES)]

                    def norm1(v, carry, _b=b, _w1=w1):
                        vsl = pl.ds(v * LANES, LANES)
                        x = stage[vsl]
                        x = x / jnp.maximum(x, 1.0)
                        stage[vsl] = (e1w[_b][vsl] + x * _w1) * fbuf[vsl]
                        return carry
                    lax.fori_loop(0, rpt // LANES, norm1, 0)
                    pltpu.sync_copy(
                        stage, out_hbm.at[pl.ds(gb * nent_pad + ent0, rpt)])
            if t == 0:
                plsc.subcore_barrier()

    return fused(e_flat, sub, obj, p_flat, w_flat, f_flat)


def kernel(e_s, pair, d_prob, hop_attn_logits, q_mask_logits):
    num_steps, bsz, E = d_prob.shape
    num_ent = e_s.shape[1]

    nent_pad = -(-num_ent // (NS * LANES)) * (NS * LANES)
    e_pad = -(-E // (NS * CHUNK)) * (NS * CHUNK)
    ept = e_pad // NS

    sub = pair[:, 0]
    obj = pair[:, 1]
    if e_pad != E:
        sub = jnp.pad(sub, (0, e_pad - E))
        obj = jnp.pad(obj, (0, e_pad - E))
        d_prob = jnp.pad(d_prob, ((0, 0), (0, 0), (0, e_pad - E)))

    e_flat = jnp.pad(e_s, ((0, 0), (0, nent_pad - num_ent))).reshape(-1)

    hop_attn = jax.nn.softmax(hop_attn_logits, axis=1)
    w_flat = jnp.broadcast_to(hop_attn.T[:, :, None],
                              (num_steps, bsz, LANES)).reshape(-1)
    flag = (jnp.argmax(hop_attn, axis=1) == 1).astype(jnp.float32)[:, None]
    f = (1.0 - flag * e_s) * jax.nn.sigmoid(q_mask_logits)
    f_flat = jnp.pad(f, ((0, 0), (0, nent_pad - num_ent))).reshape(-1)

    out = _fused_call(e_flat, sub, obj, d_prob.reshape(-1), w_flat, f_flat,
                      nent_pad, ept, e_pad)
    return out.reshape(bsz, nent_pad)[:, :num_ent]


# speedup vs baseline: 1.1455x; 1.1455x over previous
import functools

import jax
import jax.numpy as jnp
from jax import lax
from jax.experimental import pallas as pl
from jax.experimental.pallas import tpu as pltpu
from jax.experimental.pallas import tpu_sc as plsc

NC = 2
NS = 16
NW = NC * NS
LANES = 16
BSZ = 4
NPAIR = BSZ // 2

CHUNK = 2000


@functools.partial(jax.jit, static_argnums=(5, 6))
def _step_call(tab_packed, sub, obj, p, zeros, nent_pad, ept):
    e_pad = sub.shape[0]
    nchunk = ept // CHUNK
    rpt = nent_pad // NS

    mesh = plsc.VectorSubcoreMesh(
        core_axis_name="c", subcore_axis_name="s",
        num_cores=NC, num_subcores=NS)

    @functools.partial(
        pl.kernel,
        out_type=jax.ShapeDtypeStruct((NC * BSZ * nent_pad,), jnp.float32),
        mesh=mesh,
        scratch_types=(
            [pltpu.VMEM_SHARED((nent_pad,), jnp.int32)
             for _ in range(NPAIR)]
            + [pltpu.VMEM_SHARED((nent_pad,), jnp.float32)
               for _ in range(BSZ)]
            + [pltpu.VMEM((CHUNK,), jnp.int32) for _ in range(4)]
            + [pltpu.VMEM((CHUNK,), jnp.int32) for _ in range(2 * NPAIR)]
            + [pltpu.VMEM((CHUNK,), jnp.float32) for _ in range(4 * BSZ)]
            + [pltpu.VMEM((rpt,), jnp.float32)]
            + [pltpu.VMEM((rpt,), jnp.int32)]
            + [pltpu.SemaphoreType.DMA for _ in range(5)]
        ),
    )
    def step(tab_hbm, sub_hbm, obj_hbm, p_hbm, zeros_hbm, out_hbm,
             tp0, tp1, a0, a1, a2, a3,
             sub0, sub1, obj0, obj1,
             gp00, gp01, gp10, gp11,
             pc00, pc01, pc02, pc03, pc10, pc11, pc12, pc13,
             m00, m01, m02, m03, m10, m11, m12, m13,
             stage, istage,
             sin0, sin1, sg, ss0, ss1):
        tabs = (tp0, tp1)
        accs = (a0, a1, a2, a3)
        subc = (sub0, sub1)
        objc = (obj0, obj1)
        gps = ((gp00, gp01), (gp10, gp11))
        pcs = ((pc00, pc01, pc02, pc03), (pc10, pc11, pc12, pc13))
        msgs = ((m00, m01, m02, m03), (m10, m11, m12, m13))
        sem_in = (sin0, sin1)
        sem_s = (ss0, ss1)
        c = lax.axis_index("c")
        s = lax.axis_index("s")
        wid = c * NS + s

        def issue_inputs(j, slot):
            base = wid * ept + j * CHUNK
            pltpu.async_copy(sub_hbm.at[pl.ds(base, CHUNK)], subc[slot],
                             sem_in[slot])
            pltpu.async_copy(obj_hbm.at[pl.ds(base, CHUNK)], objc[slot],
                             sem_in[slot])
            for b in range(BSZ):
                pltpu.async_copy(p_hbm.at[pl.ds(b * e_pad + base, CHUNK)],
                                 pcs[slot][b], sem_in[slot])

        def wait_inputs(slot):
            pltpu.make_async_copy(sub_hbm.at[pl.ds(0, CHUNK)], subc[slot],
                                  sem_in[slot]).wait()
            pltpu.make_async_copy(obj_hbm.at[pl.ds(0, CHUNK)], objc[slot],
                                  sem_in[slot]).wait()
            for b in range(BSZ):
                pltpu.make_async_copy(p_hbm.at[pl.ds(0, CHUNK)],
                                      pcs[slot][b], sem_in[slot]).wait()

        def wait_scatters(slot):
            for b in range(BSZ):
                pltpu.make_async_copy(msgs[slot][b], accs[b].at[objc[slot]],
                                      sem_s[slot]).wait()

        ent0 = s * rpt
        pltpu.sync_copy(zeros_hbm.at[pl.ds(ent0, rpt)], stage)
        for b in range(BSZ):
            pltpu.sync_copy(stage, accs[b].at[pl.ds(ent0, rpt)])
        for j in range(NPAIR):
            pltpu.sync_copy(tab_hbm.at[pl.ds(j * nent_pad + ent0, rpt)],
                            istage)
            pltpu.sync_copy(istage, tabs[j].at[pl.ds(ent0, rpt)])
        plsc.subcore_barrier()

        issue_inputs(0, 0)

        def chunk_body(i, carry):
            slot = lax.rem(i, 2)

            def slot_body(sl, ot):
                @pl.when(i > 0)
                def _():
                    wait_scatters(ot)
                nxt = jnp.minimum(i + 1, nchunk - 1)
                issue_inputs(nxt, ot)
                wait_inputs(sl)
                for j in range(NPAIR):
                    pltpu.async_copy(tabs[j].at[subc[sl]], gps[sl][j], sg)
                for j in range(NPAIR):
                    pltpu.make_async_copy(tabs[j].at[subc[sl]],
                                          gps[sl][j], sg).wait()
                def mul_body(v, carry2):
                    vsl = pl.ds(v * LANES, LANES)
                    for j in range(NPAIR):
                        packed = gps[sl][j][vsl]
                        ga = lax.bitcast_convert_type(
                            lax.shift_left(packed, 16), jnp.float32)
                        gb = lax.bitcast_convert_type(
                            lax.bitwise_and(packed, jnp.int32(-65536)),
                            jnp.float32)
                        msgs[sl][2 * j][vsl] = ga * pcs[sl][2 * j][vsl]
                        msgs[sl][2 * j + 1][vsl] = (
                            gb * pcs[sl][2 * j + 1][vsl])
                    return carry2
                lax.fori_loop(0, CHUNK // LANES, mul_body, 0)
                for b in range(BSZ):
                    pltpu.async_copy(msgs[sl][b], accs[b].at[objc[sl]],
                                     sem_s[sl], add=True)

            @pl.when(slot == 0)
            def _():
                slot_body(0, 1)

            @pl.when(slot == 1)
            def _():
                slot_body(1, 0)
            return carry
        lax.fori_loop(0, nchunk, chunk_body, 0)

        wait_scatters((nchunk - 1) % 2)
        wait_inputs(nchunk % 2)

        plsc.subcore_barrier()
        for b in range(BSZ):
            pltpu.sync_copy(accs[b].at[pl.ds(ent0, rpt)], stage)
            pltpu.sync_copy(
                stage,
                out_hbm.at[pl.ds((c * BSZ + b) * nent_pad + ent0, rpt)])

    return step(tab_packed, sub, obj, p, zeros)


def _pack_pairs(e):
    pairs = []
    for j in range(NPAIR):
        st = jnp.stack([e[2 * j], e[2 * j + 1]], axis=-1)
        pairs.append(lax.bitcast_convert_type(
            st.astype(jnp.bfloat16), jnp.int32))
    return jnp.concatenate(pairs)


def kernel(e_s, pair, d_prob, hop_attn_logits, q_mask_logits):
    num_steps, bsz, E = d_prob.shape
    num_ent = e_s.shape[1]

    nent_pad = -(-num_ent // (NS * LANES)) * (NS * LANES)
    e_pad = -(-E // (NW * CHUNK)) * (NW * CHUNK)
    ept = e_pad // NW

    sub = pair[:, 0]
    obj = pair[:, 1]
    if e_pad != E:
        sub = jnp.pad(sub, (0, e_pad - E))
        obj = jnp.pad(obj, (0, e_pad - E))
        d_prob = jnp.pad(d_prob, ((0, 0), (0, 0), (0, e_pad - E)))

    zeros = jnp.zeros((nent_pad,), jnp.float32)
    tab = jnp.pad(e_s, ((0, 0), (0, nent_pad - num_ent)))

    ent_probs = []
    for t in range(num_steps):
        parts = _step_call(_pack_pairs(tab), sub, obj,
                           d_prob[t].reshape(-1), zeros, nent_pad, ept)
        parts = parts.reshape(NC, bsz, nent_pad)
        new_e = parts[0] + parts[1]
        tab = new_e / jnp.maximum(new_e, 1.0)
        ent_probs.append(tab[:, :num_ent])

    hop_attn = jax.nn.softmax(hop_attn_logits, axis=1)
    last_e = sum(ent_probs[t] * hop_attn[:, t:t + 1] for t in range(num_steps))
    m = (jnp.argmax(hop_attn, axis=1) == 1).astype(jnp.float32)[:, None] * e_s
    last_e = (1.0 - m) * last_e
    last_e = last_e * jax.nn.sigmoid(q_mask_logits)
    return last_e


# speedup vs baseline: 1.3106x; 1.1441x over previous
import functools

import jax
import jax.numpy as jnp
from jax import lax
from jax.experimental import pallas as pl
from jax.experimental.pallas import tpu as pltpu
from jax.experimental.pallas import tpu_sc as plsc

NC = 2
NS = 16
NW = NC * NS
LANES = 16
BSZ = 4
NPAIR = BSZ // 2

CHUNK = 2000


@functools.partial(jax.jit, static_argnums=(5, 6))
def _step_call(tab_packed, sub, obj, p, zeros, nent_pad, ept):
    e_pad = sub.shape[0]
    nchunk = ept // CHUNK
    rpt = nent_pad // NS

    mesh = plsc.VectorSubcoreMesh(
        core_axis_name="c", subcore_axis_name="s",
        num_cores=NC, num_subcores=NS)

    @functools.partial(
        pl.kernel,
        out_type=jax.ShapeDtypeStruct((NC * BSZ * nent_pad,), jnp.float32),
        mesh=mesh,
        scratch_types=(
            [pltpu.VMEM_SHARED((nent_pad,), jnp.int32)
             for _ in range(NPAIR)]
            + [pltpu.VMEM_SHARED((nent_pad,), jnp.float32)
               for _ in range(BSZ)]
            + [pltpu.VMEM((CHUNK,), jnp.int32) for _ in range(2)]
            + [pltpu.VMEM((CHUNK,), jnp.int32) for _ in range(4)]
            + [pltpu.VMEM((CHUNK,), jnp.int32) for _ in range(2 * NPAIR)]
            + [pltpu.VMEM((CHUNK,), jnp.float32) for _ in range(4 * BSZ)]
            + [pltpu.VMEM((rpt,), jnp.float32)]
            + [pltpu.VMEM((rpt,), jnp.int32)]
            + [pltpu.SemaphoreType.DMA for _ in range(5)]
        ),
    )
    def step(tab_hbm, sub_hbm, obj_hbm, p_hbm, zeros_hbm, out_hbm,
             tp0, tp1, a0, a1, a2, a3,
             sub0, sub1, obj0, obj1, obj2, obj3,
             gp00, gp01, gp10, gp11,
             pc00, pc01, pc02, pc03, pc10, pc11, pc12, pc13,
             m00, m01, m02, m03, m10, m11, m12, m13,
             stage, istage,
             sin0, sin1, sg, ss0, ss1):
        tabs = (tp0, tp1)
        accs = (a0, a1, a2, a3)
        subc = (sub0, sub1)
        objc = (obj0, obj1, obj2, obj3)
        gps = ((gp00, gp01), (gp10, gp11))
        pcs = ((pc00, pc01, pc02, pc03), (pc10, pc11, pc12, pc13))
        msgs = ((m00, m01, m02, m03), (m10, m11, m12, m13))
        sem_in = (sin0, sin1)
        sem_s = (ss0, ss1)
        c = lax.axis_index("c")
        s = lax.axis_index("s")
        wid = c * NS + s

        def issue_inputs(j, slot, oslot):
            base = wid * ept + j * CHUNK
            pltpu.async_copy(sub_hbm.at[pl.ds(base, CHUNK)], subc[slot],
                             sem_in[slot])
            pltpu.async_copy(obj_hbm.at[pl.ds(base, CHUNK)], objc[oslot],
                             sem_in[slot])
            for b in range(BSZ):
                pltpu.async_copy(p_hbm.at[pl.ds(b * e_pad + base, CHUNK)],
                                 pcs[slot][b], sem_in[slot])

        def wait_inputs(slot, oslot):
            pltpu.make_async_copy(sub_hbm.at[pl.ds(0, CHUNK)], subc[slot],
                                  sem_in[slot]).wait()
            pltpu.make_async_copy(obj_hbm.at[pl.ds(0, CHUNK)], objc[oslot],
                                  sem_in[slot]).wait()
            for b in range(BSZ):
                pltpu.make_async_copy(p_hbm.at[pl.ds(0, CHUNK)],
                                      pcs[slot][b], sem_in[slot]).wait()

        def wait_scatters(slot, oslot):
            for b in range(BSZ):
                pltpu.make_async_copy(msgs[slot][b],
                                      accs[b].at[objc[oslot]],
                                      sem_s[slot]).wait()

        ent0 = s * rpt
        pltpu.sync_copy(zeros_hbm.at[pl.ds(ent0, rpt)], stage)
        for b in range(BSZ):
            pltpu.sync_copy(stage, accs[b].at[pl.ds(ent0, rpt)])
        for j in range(NPAIR):
            pltpu.sync_copy(tab_hbm.at[pl.ds(j * nent_pad + ent0, rpt)],
                            istage)
            pltpu.sync_copy(istage, tabs[j].at[pl.ds(ent0, rpt)])
        plsc.subcore_barrier()

        issue_inputs(0, 0, 0)

        def chunk_body(i, carry):
            s4 = lax.rem(i, 4)

            def slot_body(q):
                sl = q % 2
                ot = 1 - sl
                osl = q
                onxt = (q + 1) % 4
                oprev = (q + 2) % 4
                nxt = jnp.minimum(i + 1, nchunk - 1)
                issue_inputs(nxt, ot, onxt)
                wait_inputs(sl, osl)
                for j in range(NPAIR):
                    pltpu.async_copy(tabs[j].at[subc[sl]], gps[sl][j], sg)
                for j in range(NPAIR):
                    pltpu.make_async_copy(tabs[j].at[subc[sl]],
                                          gps[sl][j], sg).wait()
                @pl.when(i >= 2)
                def _():
                    wait_scatters(sl, oprev)
                def mul_body(v, carry2):
                    vsl = pl.ds(v * LANES, LANES)
                    for j in range(NPAIR):
                        packed = gps[sl][j][vsl]
                        ga = lax.bitcast_convert_type(
                            lax.shift_left(packed, 16), jnp.float32)
                        gb = lax.bitcast_convert_type(
                            lax.bitwise_and(packed, jnp.int32(-65536)),
                            jnp.float32)
                        msgs[sl][2 * j][vsl] = ga * pcs[sl][2 * j][vsl]
                        msgs[sl][2 * j + 1][vsl] = (
                            gb * pcs[sl][2 * j + 1][vsl])
                    return carry2
                lax.fori_loop(0, CHUNK // LANES, mul_body, 0)
                for b in range(BSZ):
                    pltpu.async_copy(msgs[sl][b], accs[b].at[objc[osl]],
                                     sem_s[sl], add=True)

            for q in range(4):
                @pl.when(s4 == q)
                def _(q=q):
                    slot_body(q)
            return carry
        lax.fori_loop(0, nchunk, chunk_body, 0)

        assert nchunk % 2 == 0
        wait_scatters((nchunk - 2) % 2, (nchunk - 2) % 4)
        wait_scatters((nchunk - 1) % 2, (nchunk - 1) % 4)
        wait_inputs(nchunk % 2, nchunk % 4)

        plsc.subcore_barrier()
        for b in range(BSZ):
            pltpu.sync_copy(accs[b].at[pl.ds(ent0, rpt)], stage)
            pltpu.sync_copy(
                stage,
                out_hbm.at[pl.ds((c * BSZ + b) * nent_pad + ent0, rpt)])

    return step(tab_packed, sub, obj, p, zeros)


def _pack_pairs(e):
    pairs = []
    for j in range(NPAIR):
        st = jnp.stack([e[2 * j], e[2 * j + 1]], axis=-1)
        pairs.append(lax.bitcast_convert_type(
            st.astype(jnp.bfloat16), jnp.int32))
    return jnp.concatenate(pairs)


def kernel(e_s, pair, d_prob, hop_attn_logits, q_mask_logits):
    num_steps, bsz, E = d_prob.shape
    num_ent = e_s.shape[1]

    nent_pad = -(-num_ent // (NS * LANES)) * (NS * LANES)
    e_pad = -(-E // (NW * CHUNK)) * (NW * CHUNK)
    ept = e_pad // NW

    sub = pair[:, 0]
    obj = pair[:, 1]
    if e_pad != E:
        sub = jnp.pad(sub, (0, e_pad - E))
        obj = jnp.pad(obj, (0, e_pad - E))
        d_prob = jnp.pad(d_prob, ((0, 0), (0, 0), (0, e_pad - E)))

    zeros = jnp.zeros((nent_pad,), jnp.float32)
    tab = jnp.pad(e_s, ((0, 0), (0, nent_pad - num_ent)))

    ent_probs = []
    for t in range(num_steps):
        parts = _step_call(_pack_pairs(tab), sub, obj,
                           d_prob[t].reshape(-1), zeros, nent_pad, ept)
        parts = parts.reshape(NC, bsz, nent_pad)
        new_e = parts[0] + parts[1]
        tab = new_e / jnp.maximum(new_e, 1.0)
        ent_probs.append(tab[:, :num_ent])

    hop_attn = jax.nn.softmax(hop_attn_logits, axis=1)
    last_e = sum(ent_probs[t] * hop_attn[:, t:t + 1] for t in range(num_steps))
    m = (jnp.argmax(hop_attn, axis=1) == 1).astype(jnp.float32)[:, None] * e_s
    last_e = (1.0 - m) * last_e
    last_e = last_e * jax.nn.sigmoid(q_mask_logits)
    return last_e


# speedup vs baseline: 1.3153x; 1.0036x over previous
import functools

import jax
import jax.numpy as jnp
from jax import lax
from jax.experimental import pallas as pl
from jax.experimental.pallas import tpu as pltpu
from jax.experimental.pallas import tpu_sc as plsc

NC = 2
NS = 16
NW = NC * NS
LANES = 16
BSZ = 4
NPAIR = BSZ // 2

CHUNK = 2000


@functools.partial(jax.jit, static_argnums=(5, 6))
def _step_call(tab_packed, sub, obj, p, zeros, nent_pad, ept):
    e_pad = sub.shape[0]
    nchunk = ept // CHUNK
    rpt = nent_pad // NS

    mesh = plsc.VectorSubcoreMesh(
        core_axis_name="c", subcore_axis_name="s",
        num_cores=NC, num_subcores=NS)

    @functools.partial(
        pl.kernel,
        out_type=jax.ShapeDtypeStruct((NC * BSZ * nent_pad,), jnp.float32),
        mesh=mesh,
        scratch_types=(
            [pltpu.VMEM_SHARED((nent_pad,), jnp.int32)
             for _ in range(NPAIR)]
            + [pltpu.VMEM_SHARED((nent_pad,), jnp.float32)
               for _ in range(BSZ)]
            + [pltpu.VMEM((CHUNK,), jnp.int32) for _ in range(2)]
            + [pltpu.VMEM((CHUNK,), jnp.int32) for _ in range(4)]
            + [pltpu.VMEM((CHUNK,), jnp.int32) for _ in range(2 * NPAIR)]
            + [pltpu.VMEM((CHUNK,), jnp.float32) for _ in range(4 * BSZ)]
            + [pltpu.VMEM((rpt,), jnp.float32)]
            + [pltpu.VMEM((rpt,), jnp.int32)]
            + [pltpu.SemaphoreType.DMA for _ in range(5)]
        ),
    )
    def step(tab_hbm, sub_hbm, obj_hbm, p_hbm, zeros_hbm, out_hbm,
             tp0, tp1, a0, a1, a2, a3,
             sub0, sub1, obj0, obj1, obj2, obj3,
             gp00, gp01, gp10, gp11,
             pc00, pc01, pc02, pc03, pc10, pc11, pc12, pc13,
             m00, m01, m02, m03, m10, m11, m12, m13,
             stage, istage,
             sin0, sin1, sg, ss0, ss1):
        tabs = (tp0, tp1)
        accs = (a0, a1, a2, a3)
        subc = (sub0, sub1)
        objc = (obj0, obj1, obj2, obj3)
        gps = ((gp00, gp01), (gp10, gp11))
        pcs = ((pc00, pc01, pc02, pc03), (pc10, pc11, pc12, pc13))
        msgs = ((m00, m01, m02, m03), (m10, m11, m12, m13))
        sem_in = (sin0, sin1)
        sem_s = (ss0, ss1)
        c = lax.axis_index("c")
        s = lax.axis_index("s")
        wid = c * NS + s

        def issue_inputs(j, slot, oslot):
            base = wid * ept + j * CHUNK
            pltpu.async_copy(sub_hbm.at[pl.ds(base, CHUNK)], subc[slot],
                             sem_in[slot])
            pltpu.async_copy(obj_hbm.at[pl.ds(base, CHUNK)], objc[oslot],
                             sem_in[slot])
            for b in range(BSZ):
                pltpu.async_copy(p_hbm.at[pl.ds(b * e_pad + base, CHUNK)],
                                 pcs[slot][b], sem_in[slot])

        def wait_inputs(slot, oslot):
            pltpu.make_async_copy(sub_hbm.at[pl.ds(0, CHUNK)], subc[slot],
                                  sem_in[slot]).wait()
            pltpu.make_async_copy(obj_hbm.at[pl.ds(0, CHUNK)], objc[oslot],
                                  sem_in[slot]).wait()
            for b in range(BSZ):
                pltpu.make_async_copy(p_hbm.at[pl.ds(0, CHUNK)],
                                      pcs[slot][b], sem_in[slot]).wait()

        def wait_scatters(slot, oslot):
            for b in range(BSZ):
                pltpu.make_async_copy(msgs[slot][b],
                                      accs[b].at[objc[oslot]],
                                      sem_s[slot]).wait()

        ent0 = s * rpt
        pltpu.sync_copy(zeros_hbm.at[pl.ds(ent0, rpt)], stage)
        for b in range(BSZ):
            pltpu.sync_copy(stage, accs[b].at[pl.ds(ent0, rpt)])
        for j in range(NPAIR):
            pltpu.sync_copy(tab_hbm.at[pl.ds(j * nent_pad + ent0, rpt)],
                            istage)
            pltpu.sync_copy(istage, tabs[j].at[pl.ds(ent0, rpt)])
        plsc.subcore_barrier()

        issue_inputs(0, 0, 0)

        def chunk_body(i, carry):
            s4 = lax.rem(i, 4)

            def slot_body(q):
                sl = q % 2
                ot = 1 - sl
                osl = q
                onxt = (q + 1) % 4
                oprev = (q + 2) % 4
                nxt = jnp.minimum(i + 1, nchunk - 1)
                issue_inputs(nxt, ot, onxt)
                wait_inputs(sl, osl)
                for j in range(NPAIR):
                    pltpu.async_copy(tabs[j].at[subc[sl]], gps[sl][j], sg)
                for j in range(NPAIR):
                    pltpu.make_async_copy(tabs[j].at[subc[sl]],
                                          gps[sl][j], sg).wait()
                @pl.when(i >= 2)
                def _():
                    wait_scatters(sl, oprev)
                def mul_body(v, carry2):
                    vsl = pl.ds(v * LANES, LANES)
                    for j in range(NPAIR):
                        packed = gps[sl][j][vsl]
                        ga = lax.bitcast_convert_type(
                            lax.shift_left(packed, 16), jnp.float32)
                        gb = lax.bitcast_convert_type(
                            lax.bitwise_and(packed, jnp.int32(-65536)),
                            jnp.float32)
                        msgs[sl][2 * j][vsl] = ga * pcs[sl][2 * j][vsl]
                        msgs[sl][2 * j + 1][vsl] = (
                            gb * pcs[sl][2 * j + 1][vsl])
                    return carry2
                lax.fori_loop(0, CHUNK // LANES, mul_body, 0)
                for b in range(BSZ):
                    pltpu.async_copy(msgs[sl][b], accs[b].at[objc[osl]],
                                     sem_s[sl], add=True)

            for q in range(4):
                @pl.when(s4 == q)
                def _(q=q):
                    slot_body(q)
            return carry
        lax.fori_loop(0, nchunk, chunk_body, 0)

        wait_scatters((nchunk - 2) % 2, (nchunk - 2) % 4)
        wait_scatters((nchunk - 1) % 2, (nchunk - 1) % 4)
        wait_inputs(nchunk % 2, nchunk % 4)

        plsc.subcore_barrier()
        for b in range(BSZ):
            pltpu.sync_copy(accs[b].at[pl.ds(ent0, rpt)], stage)
            pltpu.sync_copy(
                stage,
                out_hbm.at[pl.ds((c * BSZ + b) * nent_pad + ent0, rpt)])

    return step(tab_packed, sub, obj, p, zeros)


def _pack_pairs(e):
    pairs = []
    for j in range(NPAIR):
        st = jnp.stack([e[2 * j], e[2 * j + 1]], axis=-1)
        pairs.append(lax.bitcast_convert_type(
            st.astype(jnp.bfloat16), jnp.int32))
    return jnp.concatenate(pairs)


def kernel(e_s, pair, d_prob, hop_attn_logits, q_mask_logits):
    num_steps, bsz, E = d_prob.shape
    num_ent = e_s.shape[1]

    nent_pad = -(-num_ent // (NS * LANES)) * (NS * LANES)
    e_pad = -(-E // (NW * CHUNK)) * (NW * CHUNK)
    ept = e_pad // NW

    sub = pair[:, 0]
    obj = pair[:, 1]
    if e_pad != E:
        sub = jnp.pad(sub, (0, e_pad - E))
        obj = jnp.pad(obj, (0, e_pad - E))
        d_prob = jnp.pad(d_prob, ((0, 0), (0, 0), (0, e_pad - E)))

    zeros = jnp.zeros((nent_pad,), jnp.float32)
    tab = jnp.pad(e_s, ((0, 0), (0, nent_pad - num_ent)))

    ent_probs = []
    for t in range(num_steps):
        parts = _step_call(_pack_pairs(tab), sub, obj,
                           d_prob[t].reshape(-1), zeros, nent_pad, ept)
        parts = parts.reshape(NC, bsz, nent_pad)
        new_e = parts[0] + parts[1]
        tab = new_e / jnp.maximum(new_e, 1.0)
        ent_probs.append(tab[:, :num_ent])

    hop_attn = jax.nn.softmax(hop_attn_logits, axis=1)
    last_e = sum(ent_probs[t] * hop_attn[:, t:t + 1] for t in range(num_steps))
    m = (jnp.argmax(hop_attn, axis=1) == 1).astype(jnp.float32)[:, None] * e_s
    last_e = (1.0 - m) * last_e
    last_e = last_e * jax.nn.sigmoid(q_mask_logits)
    return last_e
